# baseline (speedup 1.0000x reference)
import functools

import jax
import jax.numpy as jnp
from jax import lax
from jax.experimental import pallas as pl
from jax.experimental.pallas import tpu as pltpu
from jax.experimental.pallas import tpu_sc as plsc

DIM = 32
NC = 2
NS = 16
NW = NC * NS
CHUNK = 128
G = 4


def _make_gather(B: int, T: int):
    n_blocks = B // CHUNK
    n_chunks = T * n_blocks
    chunks_per_w = n_chunks // NW
    n_groups = chunks_per_w // G
    assert n_chunks % NW == 0 and chunks_per_w % G == 0 and n_groups % 2 == 0
    mesh = plsc.VectorSubcoreMesh(core_axis_name="c", subcore_axis_name="s")

    @functools.partial(
        pl.kernel,
        out_type=jax.ShapeDtypeStruct((T, DIM // 8, n_blocks, 8, CHUNK),
                                      jnp.float32),
        mesh=mesh,
        scratch_types=[
            pltpu.VMEM((chunks_per_w, CHUNK), jnp.int32),
            pltpu.VMEM((2, G, CHUNK, DIM), jnp.float32),
            pltpu.VMEM((2, G, DIM, CHUNK), jnp.float32),
            pltpu.SemaphoreType.DMA,
            pltpu.SemaphoreType.DMA,
            pltpu.SemaphoreType.DMA,
            pltpu.SemaphoreType.DMA,
        ],
        compiler_params=pltpu.CompilerParams(use_tc_tiling_on_sc=False,
                                             needs_layout_passes=False),
    )
    def gather_kernel(idx_hbm, table_hbm, out_hbm, idx_v, gbuf, tbuf,
                      gs0, gs1, ws0, ws1):
        gsem = (gs0, gs1)
        wsem = (ws0, ws1)
        wid = lax.axis_index("s") * NC + lax.axis_index("c")
        pltpu.sync_copy(idx_hbm.at[wid], idx_v)

        def fire_group(gr, slot):
            for c in range(G):
                pltpu.async_copy(
                    table_hbm.at[idx_v.at[gr * G + c]],
                    gbuf.at[slot, c],
                    gsem[slot],
                )

        def drain_scatters(slot):
            for _ in range(G * (DIM // 8)):
                pltpu.make_async_copy(
                    tbuf.at[slot, 0, pl.ds(0, 8)],
                    out_hbm.at[0, 0, 0],
                    wsem[slot],
                ).wait()

        def transpose_chunk(slot, c):
            gref = gbuf.at[slot, c]

            def dbody(d, carry):
                col = jnp.full((16,), d, jnp.int32)
                for k in range(CHUNK // 16):
                    row = lax.iota(jnp.int32, 16) + (16 * k)
                    v = plsc.load_gather(gref, [row, col])
                    tbuf[slot, c, d, pl.ds(16 * k, 16)] = v
                return carry

            lax.fori_loop(0, DIM, dbody, 0)

        def process_group(gr, slot, drain):
            for c in range(G):
                pltpu.make_async_copy(
                    table_hbm.at[idx_v.at[gr * G + c]],
                    gbuf.at[slot, c],
                    gsem[slot],
                ).wait()
            if drain:
                drain_scatters(slot)
            for c in range(G):
                cid = wid * chunks_per_w + gr * G + c
                t = cid // n_blocks
                blk = cid - t * n_blocks
                transpose_chunk(slot, c)
                for g in range(DIM // 8):
                    pltpu.async_copy(
                        tbuf.at[slot, c, pl.ds(g * 8, 8)],
                        out_hbm.at[t, g, blk],
                        wsem[slot],
                    )

        fire_group(0, 0)
        fire_group(1, 1)
        process_group(0, 0, drain=False)
        fire_group(2, 0)
        process_group(1, 1, drain=False)
        fire_group(3, 1)

        def body(i, carry):
            gr = 2 + 2 * i
            process_group(gr, 0, drain=True)
            fire_group(gr + 2, 0)
            process_group(gr + 1, 1, drain=True)
            fire_group(gr + 3, 1)
            return carry

        lax.fori_loop(0, n_groups // 2 - 2, body, 0)

        process_group(n_groups - 2, 0, drain=True)
        process_group(n_groups - 1, 1, drain=True)
        drain_scatters(0)
        drain_scatters(1)

    return gather_kernel


def kernel(x, E):
    b, t = x.shape
    idx = jnp.transpose(x).reshape(NW, (b * t) // (NW * CHUNK), CHUNK)
    out5 = _make_gather(b, t)(idx.astype(jnp.int32), E)
    return out5.transpose(2, 4, 0, 1, 3).reshape(b, t, DIM)
